# baseline (speedup 1.0000x reference)
import jax
import jax.numpy as jnp
from jax.experimental import pallas as pl
from jax.experimental.pallas import tpu as pltpu

_SIGN = -2147483648
_IMAX = 2147483647


def _key_kernel(x_ref, e_ref, key_ref):
    t = jnp.exp(x_ref[...])
    bits = jax.lax.bitcast_convert_type(t, jnp.int32)
    key = (bits << 1) ^ _SIGN
    key_ref[...] = jnp.where(e_ref[...] > 0.0, key, _IMAX)


def _count_kernel(xb_ref, eb_ref, sb_ref, ka_ref, sa_ref, out_ref,
                  kc_ref, sc_ref):
    rows, cols = xb_ref.shape
    n = rows * cols
    t_b = jnp.exp(xb_ref[...])
    bits_b = jax.lax.bitcast_convert_type(t_b, jnp.int32)
    e_b = eb_ref[...].astype(jnp.int32)
    key_b = ((bits_b << 1) | (1 - e_b)) ^ _SIGN
    s_b = sb_ref[...]

    nsl = 8
    sl = n // nsl
    reg = sl + 8

    def compact(i, cnts):
        new = []
        for s in range(nsl):
            a = s * sl + i
            ka = ka_ref[a]
            kc_ref[s * reg + cnts[s]] = ka
            sc_ref[s * reg + cnts[s]] = sa_ref[a]
            new.append(cnts[s] + jnp.where(ka != _IMAX, 1, 0).astype(jnp.int32))
        return tuple(new)

    cnts = jax.lax.fori_loop(0, sl, compact, (jnp.int32(0),) * nsl)

    for s in range(nsl):
        for j in range(8):
            kc_ref[s * reg + cnts[s] + j] = _IMAX

    def make_body(s):
        def body(i, carry):
            acc1, acc2 = carry
            base = s * reg + i * 8
            for j in range(8):
                ka = kc_ref[base + j]
                sa = sc_ref[base + j]
                cmp = ka < key_b
                conm = s_b < sa
                tiem = jnp.abs(s_b - sa) <= 1e-8
                acc1 = acc1 + jnp.where(cmp, jnp.where(conm, 65537, 1), 0)
                acc2 = acc2 + jnp.where(cmp & tiem, 1, 0)
            return acc1, acc2
        return body

    zeros = jnp.zeros((rows, cols), jnp.int32)
    carry = (zeros, zeros)
    for s in range(nsl):
        carry = jax.lax.fori_loop(0, (cnts[s] + 7) // 8, make_body(s), carry)
    acc1, acc2 = carry

    tot = jnp.sum(acc1 & 65535)
    con = jnp.sum(acc1 >> 16)
    tie = jnp.sum(acc2)

    tie_f = tie.astype(jnp.float32)
    tot_f = tot.astype(jnp.float32)
    disc_f = (tot - con - tie).astype(jnp.float32)
    loss = 1.0 - (disc_f + 0.5 * tie_f) / (tot_f + 1e-7)
    out_ref[...] = jnp.broadcast_to(loss, (1, 1))


def kernel(event_indicator, event_time, estimate):
    x = jnp.asarray(event_time, jnp.float32).reshape(-1)
    s = jnp.asarray(estimate, jnp.float32).reshape(-1)
    e = jnp.asarray(event_indicator).astype(jnp.float32).reshape(-1)
    n = x.shape[0]
    rows, cols = 8, n // 8

    key_a = pl.pallas_call(
        _key_kernel,
        out_shape=jax.ShapeDtypeStruct((1, n), jnp.int32),
    )(x.reshape(1, n), e.reshape(1, n))

    out = pl.pallas_call(
        _count_kernel,
        in_specs=[
            pl.BlockSpec(memory_space=pltpu.VMEM),
            pl.BlockSpec(memory_space=pltpu.VMEM),
            pl.BlockSpec(memory_space=pltpu.VMEM),
            pl.BlockSpec(memory_space=pltpu.SMEM),
            pl.BlockSpec(memory_space=pltpu.SMEM),
        ],
        out_shape=jax.ShapeDtypeStruct((1, 1), jnp.float32),
        scratch_shapes=[
            pltpu.SMEM((n + 64,), jnp.int32),
            pltpu.SMEM((n + 64,), jnp.float32),
        ],
    )(
        x.reshape(rows, cols), e.reshape(rows, cols), s.reshape(rows, cols),
        key_a.reshape(n), s.reshape(n),
    )
    return out[0, 0]


# speedup vs baseline: 1.1538x; 1.1538x over previous
import jax
import jax.numpy as jnp
from jax.experimental import pallas as pl
from jax.experimental.pallas import tpu as pltpu

_IMAX = 2147483647
_SEG = 512
_UNROLL = 8


def _key_kernel(x_ref, e_ref, key_ref):
    t = jnp.exp(x_ref[...])
    bits = jax.lax.bitcast_convert_type(t, jnp.int32)
    key_ref[...] = jnp.where(e_ref[...] > 0.0, bits, _IMAX)


def _count_kernel(xb_ref, eb_ref, sb_ref, ka_ref, sa_ref, out_ref):
    rows, cols = xb_ref.shape
    n = rows * cols
    t_b = jnp.exp(xb_ref[...])
    bits_b = jax.lax.bitcast_convert_type(t_b, jnp.int32)
    e_b = eb_ref[...].astype(jnp.int32)
    key_b = bits_b + (1 - e_b)
    s_b = sb_ref[...]

    zeros = jnp.zeros((rows, cols), jnp.int32)

    def seg_body(seg, carry):
        tot_acc, con_acc, tie_acc = carry

        def body(i, pk):
            a = seg * _SEG + i
            ka = ka_ref[a]
            sa = sa_ref[a]
            cmp = ka < key_b
            conm = s_b < sa
            tiem = jnp.abs(s_b - sa) <= 1e-8
            w = jnp.where(conm, 1025, 1) + jnp.where(tiem, 1048576, 0)
            return pk + jnp.where(cmp, w, 0)

        pk = jax.lax.fori_loop(0, _SEG, body, zeros, unroll=_UNROLL)
        tot_acc = tot_acc + (pk & 1023)
        con_acc = con_acc + ((pk >> 10) & 1023)
        tie_acc = tie_acc + (pk >> 20)
        return tot_acc, con_acc, tie_acc

    tot_acc, con_acc, tie_acc = jax.lax.fori_loop(
        0, n // _SEG, seg_body, (zeros, zeros, zeros))

    tot = jnp.sum(tot_acc)
    con = jnp.sum(con_acc)
    tie = jnp.sum(tie_acc)

    tie_f = tie.astype(jnp.float32)
    tot_f = tot.astype(jnp.float32)
    disc_f = (tot - con - tie).astype(jnp.float32)
    loss = 1.0 - (disc_f + 0.5 * tie_f) / (tot_f + 1e-7)
    out_ref[...] = jnp.broadcast_to(loss, (1, 1))


def kernel(event_indicator, event_time, estimate):
    x = jnp.asarray(event_time, jnp.float32).reshape(-1)
    s = jnp.asarray(estimate, jnp.float32).reshape(-1)
    e = jnp.asarray(event_indicator).astype(jnp.float32).reshape(-1)
    n = x.shape[0]
    rows, cols = 8, n // 8

    key_a = pl.pallas_call(
        _key_kernel,
        out_shape=jax.ShapeDtypeStruct((1, n), jnp.int32),
    )(x.reshape(1, n), e.reshape(1, n))

    out = pl.pallas_call(
        _count_kernel,
        in_specs=[
            pl.BlockSpec(memory_space=pltpu.VMEM),
            pl.BlockSpec(memory_space=pltpu.VMEM),
            pl.BlockSpec(memory_space=pltpu.VMEM),
            pl.BlockSpec(memory_space=pltpu.SMEM),
            pl.BlockSpec(memory_space=pltpu.SMEM),
        ],
        out_shape=jax.ShapeDtypeStruct((1, 1), jnp.float32),
    )(
        x.reshape(rows, cols), e.reshape(rows, cols), s.reshape(rows, cols),
        key_a.reshape(n), s.reshape(n),
    )
    return out[0, 0]


# speedup vs baseline: 1.1802x; 1.0228x over previous
import jax
import jax.numpy as jnp
from jax.experimental import pallas as pl
from jax.experimental.pallas import tpu as pltpu

_IMIN = -2147483648
_IMAX = 2147483647


def _count_kernel(xb_ref, eb_ref, sb_ref, kab_ref, ea_ref, sa_ref, out_ref):
    rows, cols = xb_ref.shape
    n = rows * cols
    bits_b = jax.lax.bitcast_convert_type(xb_ref[...], jnp.int32)
    ord_b = jnp.where(bits_b >= 0, bits_b, _IMIN - bits_b)
    e_b = eb_ref[...].astype(jnp.int32)
    key_b = ord_b + (1 - e_b)
    s_b = sb_ref[...]

    def body(a, carry):
        acc1, acc2 = carry
        ba = kab_ref[a]
        orda = jnp.where(ba >= 0, ba, _IMIN - ba)
        ka = jnp.where(ea_ref[a] > 0, orda, _IMAX)
        sa = sa_ref[a]
        cmp = ka < key_b
        conm = s_b < sa
        tiem = jnp.abs(s_b - sa) <= 1e-8
        acc1 = acc1 + jnp.where(cmp, jnp.where(conm, 65537, 1), 0)
        acc2 = acc2 + jnp.where(cmp & tiem, 1, 0)
        return acc1, acc2

    zeros = jnp.zeros((rows, cols), jnp.int32)
    acc1, acc2 = jax.lax.fori_loop(0, n, body, (zeros, zeros), unroll=8)

    tot = jnp.sum(acc1 & 65535)
    con = jnp.sum(acc1 >> 16)
    tie = jnp.sum(acc2)

    tie_f = tie.astype(jnp.float32)
    tot_f = tot.astype(jnp.float32)
    disc_f = (tot - con - tie).astype(jnp.float32)
    loss = 1.0 - (disc_f + 0.5 * tie_f) / (tot_f + 1e-7)
    out_ref[...] = jnp.broadcast_to(loss, (1, 1))


def kernel(event_indicator, event_time, estimate):
    x = jnp.asarray(event_time, jnp.float32).reshape(-1)
    s = jnp.asarray(estimate, jnp.float32).reshape(-1)
    e = jnp.asarray(event_indicator).astype(jnp.int32).reshape(-1)
    xbits = jax.lax.bitcast_convert_type(x, jnp.int32)
    n = x.shape[0]
    rows, cols = 8, n // 8

    out = pl.pallas_call(
        _count_kernel,
        in_specs=[
            pl.BlockSpec(memory_space=pltpu.VMEM),
            pl.BlockSpec(memory_space=pltpu.VMEM),
            pl.BlockSpec(memory_space=pltpu.VMEM),
            pl.BlockSpec(memory_space=pltpu.SMEM),
            pl.BlockSpec(memory_space=pltpu.SMEM),
            pl.BlockSpec(memory_space=pltpu.SMEM),
        ],
        out_shape=jax.ShapeDtypeStruct((1, 1), jnp.float32),
    )(
        x.reshape(rows, cols),
        e.astype(jnp.float32).reshape(rows, cols),
        s.reshape(rows, cols),
        xbits, e, s,
    )
    return out[0, 0]


# speedup vs baseline: 1.2410x; 1.0516x over previous
import jax
import jax.numpy as jnp
from jax.experimental import pallas as pl
from jax.experimental.pallas import tpu as pltpu

_IMIN = -2147483648
_IMAX = 2147483647


def _count_kernel(xb_ref, eb_ref, sb_ref, kab_ref, ea_ref, sa_ref, out_ref):
    rows, cols = xb_ref.shape
    n = rows * cols
    bits_b = jax.lax.bitcast_convert_type(xb_ref[...], jnp.int32)
    ord_b = jnp.where(bits_b >= 0, bits_b, _IMIN - bits_b)
    e_b = eb_ref[...].astype(jnp.int32)
    key_b = ord_b + (1 - e_b)
    s_b = sb_ref[...]

    def body(a, carry):
        acc1, acc2 = carry
        ba = kab_ref[a]
        orda = jnp.where(ba >= 0, ba, _IMIN - ba)
        ka = jnp.where(ea_ref[a] > 0, orda, _IMAX)
        sa = sa_ref[a]
        cmp = ka < key_b
        conm = s_b < sa
        tiem = jnp.abs(s_b - sa) <= 1e-8
        acc1 = acc1 + jnp.where(cmp, jnp.where(conm, 65537, 1), 0)
        acc2 = acc2 + jnp.where(cmp & tiem, 1, 0)
        return acc1, acc2

    zeros = jnp.zeros((rows, cols), jnp.int32)
    acc1, acc2 = jax.lax.fori_loop(0, n, body, (zeros, zeros), unroll=16)

    tot = jnp.sum(acc1 & 65535)
    con = jnp.sum(acc1 >> 16)
    tie = jnp.sum(acc2)

    tie_f = tie.astype(jnp.float32)
    tot_f = tot.astype(jnp.float32)
    disc_f = (tot - con - tie).astype(jnp.float32)
    loss = 1.0 - (disc_f + 0.5 * tie_f) / (tot_f + 1e-7)
    out_ref[...] = jnp.broadcast_to(loss, (1, 1))


def kernel(event_indicator, event_time, estimate):
    x = jnp.asarray(event_time, jnp.float32).reshape(-1)
    s = jnp.asarray(estimate, jnp.float32).reshape(-1)
    e = jnp.asarray(event_indicator).astype(jnp.int32).reshape(-1)
    xbits = jax.lax.bitcast_convert_type(x, jnp.int32)
    n = x.shape[0]
    rows, cols = 8, n // 8

    out = pl.pallas_call(
        _count_kernel,
        in_specs=[
            pl.BlockSpec(memory_space=pltpu.VMEM),
            pl.BlockSpec(memory_space=pltpu.VMEM),
            pl.BlockSpec(memory_space=pltpu.VMEM),
            pl.BlockSpec(memory_space=pltpu.SMEM),
            pl.BlockSpec(memory_space=pltpu.SMEM),
            pl.BlockSpec(memory_space=pltpu.SMEM),
        ],
        out_shape=jax.ShapeDtypeStruct((1, 1), jnp.float32),
    )(
        x.reshape(rows, cols),
        e.astype(jnp.float32).reshape(rows, cols),
        s.reshape(rows, cols),
        xbits, e, s,
    )
    return out[0, 0]


# speedup vs baseline: 1.2753x; 1.0277x over previous
import jax
import jax.numpy as jnp
from jax.experimental import pallas as pl
from jax.experimental.pallas import tpu as pltpu

_IMIN = -2147483648
_IMAX = 2147483647


def _count_kernel(xb_ref, eb_ref, sb_ref, kab_ref, ea_ref, sa_ref, out_ref):
    rows, cols = xb_ref.shape
    n = rows * cols
    bits_b = jax.lax.bitcast_convert_type(xb_ref[...], jnp.int32)
    ord_b = jnp.where(bits_b >= 0, bits_b, _IMIN - bits_b)
    e_b = eb_ref[...].astype(jnp.int32)
    key_b = ord_b + (1 - e_b)
    s_b = sb_ref[...]

    def body(a, carry):
        acc1, acc2 = carry
        ba = kab_ref[a]
        orda = jnp.where(ba >= 0, ba, _IMIN - ba)
        ka = jnp.where(ea_ref[a] > 0, orda, _IMAX)
        sa = sa_ref[a]
        cmp = ka < key_b
        conm = s_b < sa
        tiem = jnp.abs(s_b - sa) <= 1e-8
        acc1 = acc1 + jnp.where(cmp, jnp.where(conm, 65537, 1), 0)
        acc2 = acc2 + jnp.where(cmp & tiem, 1, 0)
        return acc1, acc2

    zeros = jnp.zeros((rows, cols), jnp.int32)
    acc1, acc2 = jax.lax.fori_loop(0, n, body, (zeros, zeros), unroll=32)

    tot = jnp.sum(acc1 & 65535)
    con = jnp.sum(acc1 >> 16)
    tie = jnp.sum(acc2)

    tie_f = tie.astype(jnp.float32)
    tot_f = tot.astype(jnp.float32)
    disc_f = (tot - con - tie).astype(jnp.float32)
    loss = 1.0 - (disc_f + 0.5 * tie_f) / (tot_f + 1e-7)
    out_ref[...] = jnp.broadcast_to(loss, (1, 1))


def kernel(event_indicator, event_time, estimate):
    x = jnp.asarray(event_time, jnp.float32).reshape(-1)
    s = jnp.asarray(estimate, jnp.float32).reshape(-1)
    e = jnp.asarray(event_indicator).astype(jnp.int32).reshape(-1)
    xbits = jax.lax.bitcast_convert_type(x, jnp.int32)
    n = x.shape[0]
    rows, cols = 8, n // 8

    out = pl.pallas_call(
        _count_kernel,
        in_specs=[
            pl.BlockSpec(memory_space=pltpu.VMEM),
            pl.BlockSpec(memory_space=pltpu.VMEM),
            pl.BlockSpec(memory_space=pltpu.VMEM),
            pl.BlockSpec(memory_space=pltpu.SMEM),
            pl.BlockSpec(memory_space=pltpu.SMEM),
            pl.BlockSpec(memory_space=pltpu.SMEM),
        ],
        out_shape=jax.ShapeDtypeStruct((1, 1), jnp.float32),
    )(
        x.reshape(rows, cols),
        e.astype(jnp.float32).reshape(rows, cols),
        s.reshape(rows, cols),
        xbits, e, s,
    )
    return out[0, 0]


# speedup vs baseline: 1.2972x; 1.0171x over previous
import jax
import jax.numpy as jnp
from jax.experimental import pallas as pl
from jax.experimental.pallas import tpu as pltpu

_IMIN = -2147483648
_IMAX = 2147483647


def _count_kernel(xb_ref, eb_ref, sb_ref, kab_ref, ea_ref, sa_ref, out_ref):
    rows, cols = xb_ref.shape
    n = rows * cols
    bits_b = jax.lax.bitcast_convert_type(xb_ref[...], jnp.int32)
    ord_b = jnp.where(bits_b >= 0, bits_b, _IMIN - bits_b)
    e_b = eb_ref[...].astype(jnp.int32)
    key_b = ord_b + (1 - e_b)
    s_b = sb_ref[...]

    def body(a, carry):
        acc1, acc2 = carry
        ba = kab_ref[a]
        orda = jnp.where(ba >= 0, ba, _IMIN - ba)
        ka = jnp.where(ea_ref[a] > 0, orda, _IMAX)
        sa = sa_ref[a]
        cmp = ka < key_b
        conm = s_b < sa
        tiem = jnp.abs(s_b - sa) <= 1e-8
        acc1 = acc1 + jnp.where(cmp, jnp.where(conm, 65537, 1), 0)
        acc2 = acc2 + jnp.where(cmp & tiem, 1, 0)
        return acc1, acc2

    zeros = jnp.zeros((rows, cols), jnp.int32)
    acc1, acc2 = jax.lax.fori_loop(0, n, body, (zeros, zeros), unroll=64)

    tot = jnp.sum(acc1 & 65535)
    con = jnp.sum(acc1 >> 16)
    tie = jnp.sum(acc2)

    tie_f = tie.astype(jnp.float32)
    tot_f = tot.astype(jnp.float32)
    disc_f = (tot - con - tie).astype(jnp.float32)
    loss = 1.0 - (disc_f + 0.5 * tie_f) / (tot_f + 1e-7)
    out_ref[...] = jnp.broadcast_to(loss, (1, 1))


def kernel(event_indicator, event_time, estimate):
    x = jnp.asarray(event_time, jnp.float32).reshape(-1)
    s = jnp.asarray(estimate, jnp.float32).reshape(-1)
    e = jnp.asarray(event_indicator).astype(jnp.int32).reshape(-1)
    xbits = jax.lax.bitcast_convert_type(x, jnp.int32)
    n = x.shape[0]
    rows, cols = 8, n // 8

    out = pl.pallas_call(
        _count_kernel,
        in_specs=[
            pl.BlockSpec(memory_space=pltpu.VMEM),
            pl.BlockSpec(memory_space=pltpu.VMEM),
            pl.BlockSpec(memory_space=pltpu.VMEM),
            pl.BlockSpec(memory_space=pltpu.SMEM),
            pl.BlockSpec(memory_space=pltpu.SMEM),
            pl.BlockSpec(memory_space=pltpu.SMEM),
        ],
        out_shape=jax.ShapeDtypeStruct((1, 1), jnp.float32),
    )(
        x.reshape(rows, cols),
        e.astype(jnp.float32).reshape(rows, cols),
        s.reshape(rows, cols),
        xbits, e, s,
    )
    return out[0, 0]


# speedup vs baseline: 1.3086x; 1.0088x over previous
import jax
import jax.numpy as jnp
from jax.experimental import pallas as pl
from jax.experimental.pallas import tpu as pltpu

_IMIN = -2147483648
_IMAX = 2147483647


def _count_kernel(xb_ref, eb_ref, sb_ref, kab_ref, ea_ref, sa_ref, out_ref):
    rows, cols = xb_ref.shape
    n = rows * cols
    bits_b = jax.lax.bitcast_convert_type(xb_ref[...], jnp.int32)
    ord_b = jnp.where(bits_b >= 0, bits_b, _IMIN - bits_b)
    e_b = eb_ref[...].astype(jnp.int32)
    key_b = ord_b + (1 - e_b)
    s_b = sb_ref[...]

    def body(a, carry):
        acc1, acc2 = carry
        ba = kab_ref[a]
        orda = jnp.where(ba >= 0, ba, _IMIN - ba)
        ka = jnp.where(ea_ref[a] > 0, orda, _IMAX)
        sa = sa_ref[a]
        cmp = ka < key_b
        conm = s_b < sa
        tiem = jnp.abs(s_b - sa) <= 1e-8
        acc1 = acc1 + jnp.where(cmp, jnp.where(conm, 65537, 1), 0)
        acc2 = acc2 + jnp.where(cmp & tiem, 1, 0)
        return acc1, acc2

    zeros = jnp.zeros((rows, cols), jnp.int32)
    acc1, acc2 = jax.lax.fori_loop(0, n, body, (zeros, zeros), unroll=128)

    tot = jnp.sum(acc1 & 65535)
    con = jnp.sum(acc1 >> 16)
    tie = jnp.sum(acc2)

    tie_f = tie.astype(jnp.float32)
    tot_f = tot.astype(jnp.float32)
    disc_f = (tot - con - tie).astype(jnp.float32)
    loss = 1.0 - (disc_f + 0.5 * tie_f) / (tot_f + 1e-7)
    out_ref[...] = jnp.broadcast_to(loss, (1, 1))


def kernel(event_indicator, event_time, estimate):
    x = jnp.asarray(event_time, jnp.float32).reshape(-1)
    s = jnp.asarray(estimate, jnp.float32).reshape(-1)
    e = jnp.asarray(event_indicator).astype(jnp.int32).reshape(-1)
    xbits = jax.lax.bitcast_convert_type(x, jnp.int32)
    n = x.shape[0]
    rows, cols = 8, n // 8

    out = pl.pallas_call(
        _count_kernel,
        in_specs=[
            pl.BlockSpec(memory_space=pltpu.VMEM),
            pl.BlockSpec(memory_space=pltpu.VMEM),
            pl.BlockSpec(memory_space=pltpu.VMEM),
            pl.BlockSpec(memory_space=pltpu.SMEM),
            pl.BlockSpec(memory_space=pltpu.SMEM),
            pl.BlockSpec(memory_space=pltpu.SMEM),
        ],
        out_shape=jax.ShapeDtypeStruct((1, 1), jnp.float32),
    )(
        x.reshape(rows, cols),
        e.astype(jnp.float32).reshape(rows, cols),
        s.reshape(rows, cols),
        xbits, e, s,
    )
    return out[0, 0]
